# Initial kernel scaffold; baseline (speedup 1.0000x reference)
import jax
import jax.numpy as jnp
from jax.experimental import pallas as pl


def kernel(x, edge_index, deg_inv_sqrt, W1, b1, W2, b2):
    raise NotImplementedError("write your pallas kernel here")


# speedup vs baseline: 5.0127x; 5.0127x over previous
import functools

import jax
import jax.numpy as jnp
from jax import lax
from jax.experimental import pallas as pl
from jax.experimental.pallas import tpu as pltpu
from jax.experimental.pallas import tpu_sc as plsc

_N = 10000
_E = 320000
_D = 128

_NC = 2
_NS = 16
_NW = _NC * _NS
_EPW = _E // _NW
_CH = 80
_NCHUNK = _EPW // _CH
_NP = 10240
_RPT = _NP // _NS
_SR = 128
_SREP = _RPT // _SR

_sc_mesh = plsc.VectorSubcoreMesh(core_axis_name="c", subcore_axis_name="s")


@functools.partial(
    pl.kernel,
    out_type=jax.ShapeDtypeStruct((_NC, _NP, _D), jnp.float32),
    mesh=_sc_mesh,
    scratch_types=[
        pltpu.VMEM((_CH,), jnp.int32),
        pltpu.VMEM((_CH,), jnp.int32),
        pltpu.VMEM((_CH, _D), jnp.float32),
        pltpu.VMEM((_SR, _D), jnp.float32),
        pltpu.VMEM_SHARED((_NP, _D), jnp.float32),
        pltpu.SemaphoreType.DMA,
    ],
)
def _sc_aggregate(h_hbm, src_hbm, dst_hbm, zeros_hbm, out_hbm,
                  src_v, dst_v, msgs_v, stage_v, acc_sh, sem):
    cid = lax.axis_index("c")
    sid = lax.axis_index("s")
    wid = sid * _NC + cid

    pltpu.sync_copy(zeros_hbm, stage_v)
    for r in range(_SREP):
        pltpu.sync_copy(stage_v, acc_sh.at[pl.ds(sid * _RPT + r * _SR, _SR)])
    plsc.subcore_barrier()

    base = wid * _EPW

    def body(i, carry):
        off = base + i * _CH
        pltpu.sync_copy(src_hbm.at[pl.ds(off, _CH)], src_v)
        pltpu.sync_copy(dst_hbm.at[pl.ds(off, _CH)], dst_v)
        pltpu.async_copy(h_hbm.at[src_v], msgs_v, sem).wait()
        pltpu.sync_copy(msgs_v, acc_sh.at[dst_v], add=True)
        return carry

    lax.fori_loop(0, _NCHUNK, body, 0)
    plsc.subcore_barrier()

    for r in range(_SREP):
        row0 = sid * _RPT + r * _SR
        pltpu.sync_copy(acc_sh.at[pl.ds(row0, _SR)], stage_v)
        pltpu.sync_copy(stage_v, out_hbm.at[cid, pl.ds(row0, _SR)])


def _tc_pre_body(x_ref, w_ref, d_ref, o_ref):
    h = jnp.dot(x_ref[...], w_ref[...], preferred_element_type=jnp.float32)
    o_ref[...] = d_ref[...] * h


def _tc_mid_body(p_ref, d_ref, b_ref, w_ref, o_ref):
    t = d_ref[...] * (p_ref[0] + p_ref[1]) + b_ref[...]
    t = jnp.maximum(t, 0.0)
    o_ref[...] = d_ref[...] * jnp.dot(t, w_ref[...],
                                      preferred_element_type=jnp.float32)


def _tc_post_body(p_ref, d_ref, b_ref, o_ref):
    o_ref[...] = d_ref[...] * (p_ref[0] + p_ref[1]) + b_ref[...]


def kernel(x, edge_index, deg_inv_sqrt, W1, b1, W2, b2):
    src = edge_index[0]
    dst = edge_index[1]
    deg_col = deg_inv_sqrt[:, None]
    zeros = jnp.zeros((_SR, _D), jnp.float32)
    b1r = b1[None, :]
    b2r = b2[None, :]

    h1 = pl.pallas_call(
        _tc_pre_body,
        out_shape=jax.ShapeDtypeStruct((_N, _D), jnp.float32),
    )(x, W1, deg_col)

    p1 = _sc_aggregate(h1, src, dst, zeros)[:, :_N, :]

    h2 = pl.pallas_call(
        _tc_mid_body,
        out_shape=jax.ShapeDtypeStruct((_N, _D), jnp.float32),
    )(p1, deg_col, b1r, W2)

    p2 = _sc_aggregate(h2, src, dst, zeros)[:, :_N, :]

    out = pl.pallas_call(
        _tc_post_body,
        out_shape=jax.ShapeDtypeStruct((_N, _D), jnp.float32),
    )(p2, deg_col, b2r)

    return out


# speedup vs baseline: 8.9116x; 1.7778x over previous
import functools

import jax
import jax.numpy as jnp
from jax import lax
from jax.experimental import pallas as pl
from jax.experimental.pallas import tpu as pltpu
from jax.experimental.pallas import tpu_sc as plsc

_N = 10000
_E = 320000
_D = 128

_NC = 2
_NS = 16
_NW = _NC * _NS
_EPW = _E // _NW
_CH = 80
_NCHUNK = _EPW // _CH
_NP = 10112
_RPT = _NP // _NS

_sc_mesh = plsc.VectorSubcoreMesh(core_axis_name="c", subcore_axis_name="s")


@functools.partial(
    pl.kernel,
    out_type=jax.ShapeDtypeStruct((_NC, _NP, _D), jnp.float32),
    mesh=_sc_mesh,
    scratch_types=[
        [pltpu.VMEM((_CH,), jnp.int32) for _ in range(4)],
        [pltpu.VMEM((_CH,), jnp.int32) for _ in range(4)],
        [pltpu.VMEM((_CH, _D), jnp.float32) for _ in range(4)],
        pltpu.VMEM_SHARED((_NP, _D), jnp.float32),
        [pltpu.SemaphoreType.DMA for _ in range(4)],
        [pltpu.SemaphoreType.DMA for _ in range(4)],
        [pltpu.SemaphoreType.DMA for _ in range(4)],
    ],
)
def _sc_aggregate(h_hbm, src_hbm, dst_hbm, zeros_hbm, out_hbm,
                  s_ring, d_ring, m_ring, acc_sh, isems, gsems, ssems):
    cid = lax.axis_index("c")
    sid = lax.axis_index("s")
    wid = sid * _NC + cid
    row_base = sid * _RPT
    ebase = wid * _EPW

    pltpu.sync_copy(zeros_hbm, acc_sh.at[pl.ds(row_base, _RPT)])
    plsc.subcore_barrier()

    def fire_idx(c, r):
        off = ebase + c * _CH
        pltpu.async_copy(src_hbm.at[pl.ds(off, _CH)], s_ring[r], isems[r])
        pltpu.async_copy(dst_hbm.at[pl.ds(off, _CH)], d_ring[r], isems[r])

    def wait_idx(c, r):
        off = ebase + c * _CH
        pltpu.make_async_copy(src_hbm.at[pl.ds(off, _CH)], s_ring[r],
                              isems[r]).wait()
        pltpu.make_async_copy(dst_hbm.at[pl.ds(off, _CH)], d_ring[r],
                              isems[r]).wait()

    def fire_gather(r):
        pltpu.async_copy(h_hbm.at[s_ring[r]], m_ring[r], gsems[r])

    def wait_gather(r):
        pltpu.make_async_copy(h_hbm.at[s_ring[r]], m_ring[r],
                              gsems[r]).wait()

    def fire_scatter(r):
        pltpu.async_copy(m_ring[r], acc_sh.at[d_ring[r]], ssems[r], add=True)

    def wait_scatter(r):
        pltpu.make_async_copy(m_ring[r], acc_sh.at[d_ring[r]],
                              ssems[r]).wait()

    def steady(c, r):
        wait_gather(r)
        fire_scatter(r)
        wait_scatter((r + 2) % 4)
        fire_idx(c + 2, (r + 2) % 4)
        wait_idx(c + 1, (r + 1) % 4)
        fire_gather((r + 1) % 4)

    fire_idx(0, 0)
    fire_idx(1, 1)
    wait_idx(0, 0)
    fire_gather(0)
    wait_gather(0)
    fire_scatter(0)
    fire_idx(2, 2)
    wait_idx(1, 1)
    fire_gather(1)
    wait_gather(1)
    fire_scatter(1)
    fire_idx(3, 3)
    wait_idx(2, 2)
    fire_gather(2)

    def body(c, carry):
        pl.when(c % 4 == 0)(lambda: steady(c, 0))
        pl.when(c % 4 == 1)(lambda: steady(c, 1))
        pl.when(c % 4 == 2)(lambda: steady(c, 2))
        pl.when(c % 4 == 3)(lambda: steady(c, 3))
        return carry

    lax.fori_loop(2, _NCHUNK - 2, body, 0)

    wait_gather(3)
    fire_scatter(3)
    wait_scatter(1)
    wait_idx(_NCHUNK - 1, 0)
    fire_gather(0)
    wait_gather(0)
    fire_scatter(0)
    wait_scatter(2)
    wait_scatter(3)
    wait_scatter(0)
    plsc.subcore_barrier()

    pltpu.sync_copy(acc_sh.at[pl.ds(row_base, _RPT)],
                    out_hbm.at[cid, pl.ds(row_base, _RPT)])


def _tc_pre_body(x_ref, w_ref, d_ref, o_ref):
    h = jnp.dot(x_ref[...], w_ref[...], preferred_element_type=jnp.float32)
    o_ref[...] = d_ref[...] * h


def _tc_mid_body(p_ref, d_ref, b_ref, w_ref, o_ref):
    t = d_ref[...] * (p_ref[0] + p_ref[1]) + b_ref[...]
    t = jnp.maximum(t, 0.0)
    o_ref[...] = d_ref[...] * jnp.dot(t, w_ref[...],
                                      preferred_element_type=jnp.float32)


def _tc_post_body(p_ref, d_ref, b_ref, o_ref):
    o_ref[...] = d_ref[...] * (p_ref[0] + p_ref[1]) + b_ref[...]


def kernel(x, edge_index, deg_inv_sqrt, W1, b1, W2, b2):
    src = edge_index[0]
    dst = edge_index[1]
    deg_col = deg_inv_sqrt[:, None]
    zeros = jnp.zeros((_RPT, _D), jnp.float32)
    b1r = b1[None, :]
    b2r = b2[None, :]

    h1 = pl.pallas_call(
        _tc_pre_body,
        out_shape=jax.ShapeDtypeStruct((_N, _D), jnp.float32),
    )(x, W1, deg_col)

    p1 = _sc_aggregate(h1, src, dst, zeros)[:, :_N, :]

    h2 = pl.pallas_call(
        _tc_mid_body,
        out_shape=jax.ShapeDtypeStruct((_N, _D), jnp.float32),
    )(p1, deg_col, b1r, W2)

    p2 = _sc_aggregate(h2, src, dst, zeros)[:, :_N, :]

    out = pl.pallas_call(
        _tc_post_body,
        out_shape=jax.ShapeDtypeStruct((_N, _D), jnp.float32),
    )(p2, deg_col, b2r)

    return out


# speedup vs baseline: 11.2315x; 1.2603x over previous
import functools

import jax
import jax.numpy as jnp
from jax import lax
from jax.experimental import pallas as pl
from jax.experimental.pallas import tpu as pltpu
from jax.experimental.pallas import tpu_sc as plsc

_N = 10000
_E = 320000
_D = 128

_NC = 2
_NS = 16
_NW = _NC * _NS
_EPW = _E // _NW
_CH = 80
_NCHUNK = _EPW // _CH
_EPWP = _EPW
_NP = 10112
_RPT = _NP // _NS

_sc_mesh = plsc.VectorSubcoreMesh(core_axis_name="c", subcore_axis_name="s")


@functools.partial(
    pl.kernel,
    out_type=jax.ShapeDtypeStruct((_NC, _NP, _D), jnp.float32),
    mesh=_sc_mesh,
    scratch_types=[
        [pltpu.VMEM((_CH,), jnp.int32) for _ in range(4)],
        [pltpu.VMEM((_CH,), jnp.int32) for _ in range(4)],
        [pltpu.VMEM((_CH, _D), jnp.float32) for _ in range(4)],
        pltpu.VMEM_SHARED((_NP, _D), jnp.float32),
        [pltpu.SemaphoreType.DMA for _ in range(4)],
        [pltpu.SemaphoreType.DMA for _ in range(4)],
        [pltpu.SemaphoreType.DMA for _ in range(4)],
    ],
)
def _sc_aggregate(h_hbm, src_hbm, dst_hbm, zeros_hbm, out_hbm,
                  s_ring, d_ring, m_ring, acc_sh, isems, gsems, ssems):
    cid = lax.axis_index("c")
    sid = lax.axis_index("s")
    wid = sid * _NC + cid
    row_base = sid * _RPT
    ebase = wid * _EPWP

    pltpu.sync_copy(zeros_hbm, acc_sh.at[pl.ds(row_base, _RPT)])
    plsc.subcore_barrier()

    def fire_idx(c, r):
        off = ebase + c * _CH
        pltpu.async_copy(src_hbm.at[pl.ds(off, _CH)], s_ring[r], isems[r])
        pltpu.async_copy(dst_hbm.at[pl.ds(off, _CH)], d_ring[r], isems[r])

    def wait_idx(c, r):
        off = ebase + c * _CH
        pltpu.make_async_copy(src_hbm.at[pl.ds(off, _CH)], s_ring[r],
                              isems[r]).wait()
        pltpu.make_async_copy(dst_hbm.at[pl.ds(off, _CH)], d_ring[r],
                              isems[r]).wait()

    def fire_gather(r):
        pltpu.async_copy(h_hbm.at[s_ring[r]], m_ring[r], gsems[r])

    def wait_gather(r):
        pltpu.make_async_copy(h_hbm.at[s_ring[r]], m_ring[r],
                              gsems[r]).wait()

    def fire_scatter(r):
        pltpu.async_copy(m_ring[r], acc_sh.at[d_ring[r]], ssems[r], add=True)

    def wait_scatter(r):
        pltpu.make_async_copy(m_ring[r], acc_sh.at[d_ring[r]],
                              ssems[r]).wait()

    def steady(c, r):
        wait_idx(c + 1, (r + 1) % 4)
        fire_gather((r + 1) % 4)
        wait_gather(r)
        fire_scatter(r)
        wait_scatter((r + 2) % 4)
        fire_idx(c + 2, (r + 2) % 4)

    fire_idx(0, 0)
    fire_idx(1, 1)
    wait_idx(0, 0)
    fire_gather(0)
    wait_gather(0)
    fire_scatter(0)
    fire_idx(2, 2)
    wait_idx(1, 1)
    fire_gather(1)
    wait_gather(1)
    fire_scatter(1)
    fire_idx(3, 3)
    wait_idx(2, 2)
    fire_gather(2)

    def body(c, carry):
        pl.when(c % 4 == 0)(lambda: steady(c, 0))
        pl.when(c % 4 == 1)(lambda: steady(c, 1))
        pl.when(c % 4 == 2)(lambda: steady(c, 2))
        pl.when(c % 4 == 3)(lambda: steady(c, 3))
        return carry

    lax.fori_loop(2, _NCHUNK - 2, body, 0)

    wait_gather(3)
    fire_scatter(3)
    wait_scatter(1)
    wait_idx(_NCHUNK - 1, 0)
    fire_gather(0)
    wait_gather(0)
    fire_scatter(0)
    wait_scatter(2)
    wait_scatter(3)
    wait_scatter(0)
    plsc.subcore_barrier()

    pltpu.sync_copy(acc_sh.at[pl.ds(row_base, _RPT)],
                    out_hbm.at[cid, pl.ds(row_base, _RPT)])


def _tc_pre_body(x_ref, w_ref, d_ref, o_ref):
    h = jnp.dot(x_ref[...], w_ref[...], preferred_element_type=jnp.float32)
    o_ref[...] = d_ref[...] * h


def _tc_mid_body(p_ref, d_ref, b_ref, w_ref, o_ref):
    agg = p_ref[0, :_N, :] + p_ref[1, :_N, :]
    t = d_ref[...] * agg + b_ref[...]
    t = jnp.maximum(t, 0.0)
    o_ref[...] = d_ref[...] * jnp.dot(t, w_ref[...],
                                      preferred_element_type=jnp.float32)


def _tc_post_body(p_ref, d_ref, b_ref, o_ref):
    agg = p_ref[0, :_N, :] + p_ref[1, :_N, :]
    o_ref[...] = d_ref[...] * agg + b_ref[...]


def kernel(x, edge_index, deg_inv_sqrt, W1, b1, W2, b2):
    src = edge_index[0]
    dst = edge_index[1]
    deg_col = deg_inv_sqrt[:, None]
    zeros = jnp.zeros((_RPT, _D), jnp.float32)
    b1r = b1[None, :]
    b2r = b2[None, :]

    h1 = pl.pallas_call(
        _tc_pre_body,
        out_shape=jax.ShapeDtypeStruct((_N, _D), jnp.float32),
    )(x, W1, deg_col)

    p1 = _sc_aggregate(h1, src, dst, zeros)

    h2 = pl.pallas_call(
        _tc_mid_body,
        out_shape=jax.ShapeDtypeStruct((_N, _D), jnp.float32),
    )(p1, deg_col, b1r, W2)

    p2 = _sc_aggregate(h2, src, dst, zeros)

    out = pl.pallas_call(
        _tc_post_body,
        out_shape=jax.ShapeDtypeStruct((_N, _D), jnp.float32),
    )(p2, deg_col, b2r)

    return out
